# baseline (speedup 1.0000x reference)
import jax
import jax.numpy as jnp
from jax.experimental import pallas as pl
from jax.experimental.pallas import tpu as pltpu


_BLOCK_S = 2048
_NBUF = 4


def kernel(inputs, pos_table):
    batch, seq_len, out_dim = inputs.shape
    ns = seq_len // _BLOCK_S
    flat = inputs.reshape(batch * seq_len, out_dim)

    def inner(x_ref, p_ref, o_ref):
        o_ref[...] = x_ref[...] + p_ref[...]

    def outer(in_hbm, pos_hbm, o_hbm):
        pipeline = pltpu.emit_pipeline(
            inner,
            grid=(ns, batch),
            in_specs=[
                pl.BlockSpec(
                    (_BLOCK_S, out_dim),
                    lambda s, b, ns=ns: (b * ns + s, 0),
                    pipeline_mode=pl.Buffered(buffer_count=_NBUF),
                ),
                pl.BlockSpec(
                    (_BLOCK_S, out_dim),
                    lambda s, b: (s, 0),
                    pipeline_mode=pl.Buffered(buffer_count=1),
                ),
            ],
            out_specs=[
                pl.BlockSpec(
                    (_BLOCK_S, out_dim),
                    lambda s, b, ns=ns: (b * ns + s, 0),
                    pipeline_mode=pl.Buffered(buffer_count=2),
                ),
            ],
        )
        pipeline(in_hbm, pos_hbm, o_hbm)

    out = pl.pallas_call(
        outer,
        in_specs=[
            pl.BlockSpec(memory_space=pl.ANY),
            pl.BlockSpec(memory_space=pl.ANY),
        ],
        out_specs=pl.BlockSpec(memory_space=pl.ANY),
        out_shape=jax.ShapeDtypeStruct(flat.shape, flat.dtype),
    )(flat, pos_table)
    return out.reshape(batch, seq_len, out_dim)


# speedup vs baseline: 1.1089x; 1.1089x over previous
import jax
import jax.numpy as jnp
from jax.experimental import pallas as pl
from jax.experimental.pallas import tpu as pltpu


_BLOCK_S = 2048
_NBUF = 3


def kernel(inputs, pos_table):
    batch, seq_len, out_dim = inputs.shape
    ns = seq_len // _BLOCK_S
    flat = inputs.reshape(batch * seq_len, out_dim)

    def inner(x_ref, p_ref, o_ref):
        o_ref[...] = x_ref[...] + p_ref[...]

    def outer(in_hbm, pos_hbm, o_hbm):
        pipeline = pltpu.emit_pipeline(
            inner,
            grid=(ns, batch),
            in_specs=[
                pl.BlockSpec(
                    (_BLOCK_S, out_dim),
                    lambda s, b, ns=ns: (b * ns + s, 0),
                    pipeline_mode=pl.Buffered(buffer_count=_NBUF),
                ),
                pl.BlockSpec(
                    (_BLOCK_S, out_dim),
                    lambda s, b: (s, 0),
                    pipeline_mode=pl.Buffered(buffer_count=2),
                ),
            ],
            out_specs=[
                pl.BlockSpec(
                    (_BLOCK_S, out_dim),
                    lambda s, b, ns=ns: (b * ns + s, 0),
                    pipeline_mode=pl.Buffered(buffer_count=2),
                ),
            ],
        )
        pipeline(in_hbm, pos_hbm, o_hbm)

    out = pl.pallas_call(
        outer,
        in_specs=[
            pl.BlockSpec(memory_space=pl.ANY),
            pl.BlockSpec(memory_space=pl.ANY),
        ],
        out_specs=pl.BlockSpec(memory_space=pl.ANY),
        out_shape=jax.ShapeDtypeStruct(flat.shape, flat.dtype),
    )(flat, pos_table)
    return out.reshape(batch, seq_len, out_dim)


# speedup vs baseline: 1.1091x; 1.0002x over previous
import jax
import jax.numpy as jnp
from jax.experimental import pallas as pl
from jax.experimental.pallas import tpu as pltpu


_BLOCK_S = 2048
_NBUF = 3


def kernel(inputs, pos_table):
    batch, seq_len, out_dim = inputs.shape
    ns = seq_len // _BLOCK_S
    flat = inputs.reshape(batch * seq_len, out_dim)

    half = _BLOCK_S // 2

    def inner(x_ref, p_ref, o1_ref, o2_ref):
        o1_ref[...] = x_ref[pl.ds(0, half), :] + p_ref[pl.ds(0, half), :]
        o2_ref[...] = x_ref[pl.ds(half, half), :] + p_ref[pl.ds(half, half), :]

    def outer(in_hbm, pos_hbm, o_hbm):
        pipeline = pltpu.emit_pipeline(
            inner,
            grid=(ns, batch),
            in_specs=[
                pl.BlockSpec(
                    (_BLOCK_S, out_dim),
                    lambda s, b, ns=ns: (b * ns + s, 0),
                    pipeline_mode=pl.Buffered(buffer_count=_NBUF),
                ),
                pl.BlockSpec(
                    (_BLOCK_S, out_dim),
                    lambda s, b: (s, 0),
                    pipeline_mode=pl.Buffered(buffer_count=2),
                ),
            ],
            out_specs=[
                pl.BlockSpec(
                    (half, out_dim),
                    lambda s, b, ns=ns: (2 * (b * ns + s), 0),
                    pipeline_mode=pl.Buffered(buffer_count=2),
                ),
                pl.BlockSpec(
                    (half, out_dim),
                    lambda s, b, ns=ns: (2 * (b * ns + s) + 1, 0),
                    pipeline_mode=pl.Buffered(buffer_count=2),
                ),
            ],
        )
        pipeline(in_hbm, pos_hbm, o_hbm, o_hbm)

    out = pl.pallas_call(
        outer,
        in_specs=[
            pl.BlockSpec(memory_space=pl.ANY),
            pl.BlockSpec(memory_space=pl.ANY),
        ],
        out_specs=pl.BlockSpec(memory_space=pl.ANY),
        out_shape=jax.ShapeDtypeStruct(flat.shape, flat.dtype),
    )(flat, pos_table)
    return out.reshape(batch, seq_len, out_dim)


# speedup vs baseline: 1.1093x; 1.0002x over previous
import jax
import jax.numpy as jnp
from jax.experimental import pallas as pl
from jax.experimental.pallas import tpu as pltpu


_BLOCK_S = 2048
_NBUF = 3


def kernel(inputs, pos_table):
    batch, seq_len, out_dim = inputs.shape
    ns = seq_len // _BLOCK_S
    flat = inputs.reshape(batch * seq_len, out_dim)

    half = _BLOCK_S // 2

    def inner(x1_ref, x2_ref, p_ref, o1_ref, o2_ref):
        o1_ref[...] = x1_ref[...] + p_ref[pl.ds(0, half), :]
        o2_ref[...] = x2_ref[...] + p_ref[pl.ds(half, half), :]

    def outer(in_hbm, pos_hbm, o_hbm):
        pipeline = pltpu.emit_pipeline(
            inner,
            grid=(ns, batch),
            in_specs=[
                pl.BlockSpec(
                    (half, out_dim),
                    lambda s, b, ns=ns: (2 * (b * ns + s), 0),
                    pipeline_mode=pl.Buffered(buffer_count=_NBUF),
                ),
                pl.BlockSpec(
                    (half, out_dim),
                    lambda s, b, ns=ns: (2 * (b * ns + s) + 1, 0),
                    pipeline_mode=pl.Buffered(buffer_count=_NBUF),
                ),
                pl.BlockSpec(
                    (_BLOCK_S, out_dim),
                    lambda s, b: (s, 0),
                    pipeline_mode=pl.Buffered(buffer_count=2),
                ),
            ],
            out_specs=[
                pl.BlockSpec(
                    (half, out_dim),
                    lambda s, b, ns=ns: (2 * (b * ns + s), 0),
                    pipeline_mode=pl.Buffered(buffer_count=2),
                ),
                pl.BlockSpec(
                    (half, out_dim),
                    lambda s, b, ns=ns: (2 * (b * ns + s) + 1, 0),
                    pipeline_mode=pl.Buffered(buffer_count=2),
                ),
            ],
        )
        pipeline(in_hbm, in_hbm, pos_hbm, o_hbm, o_hbm)

    out = pl.pallas_call(
        outer,
        in_specs=[
            pl.BlockSpec(memory_space=pl.ANY),
            pl.BlockSpec(memory_space=pl.ANY),
        ],
        out_specs=pl.BlockSpec(memory_space=pl.ANY),
        out_shape=jax.ShapeDtypeStruct(flat.shape, flat.dtype),
    )(flat, pos_table)
    return out.reshape(batch, seq_len, out_dim)
